# baseline (speedup 1.0000x reference)
import functools

import jax
import jax.numpy as jnp
from jax.experimental import pallas as pl
from jax.experimental.pallas import tpu as pltpu

N = 10000
C = 100
NTEMPS = 350
RANKS = (0, 666, 1333, 2000, 2666, 3333, 4000, 4666, 5333, 6000,
         6666, 7333, 8000, 8666, 9333, 9999)
LO_KEY = 1008767022
HI_KEY = 1065353216
SEARCH_ITERS = 26
TEMPS_PER_STEP = 10


def _order_stats(conf):
    keys = jax.lax.bitcast_convert_type(conf, jnp.int32)
    idx = jax.lax.broadcasted_iota(jnp.int32, (16, 1), 0).astype(jnp.float32)
    ranks = jnp.minimum(jnp.floor(idx * jnp.float32(10000.0 / 15.0)),
                        jnp.float32(N - 1)).astype(jnp.int32)
    targets = ranks + 1
    ones16 = jnp.ones((16, 1), dtype=jnp.int32)
    lo = ones16 * LO_KEY
    hi = ones16 * HI_KEY

    def body(_, lohi):
        lo, hi = lohi
        mid = lo + ((hi - lo) >> 1)
        cnt = jnp.sum((keys <= mid).astype(jnp.int32), axis=1, keepdims=True)
        take = cnt >= targets
        return jnp.where(take, lo, mid), jnp.where(take, mid, hi)

    _, hi = jax.lax.fori_loop(0, SEARCH_ITERS, body, (lo, hi))
    return jax.lax.bitcast_convert_type(hi, jnp.float32)


def _order_stats_n(confs):
    keys = [jax.lax.bitcast_convert_type(c, jnp.int32) for c in confs]
    idx = jax.lax.broadcasted_iota(jnp.int32, (16, 1), 0).astype(jnp.float32)
    ranks = jnp.minimum(jnp.floor(idx * jnp.float32(10000.0 / 15.0)),
                        jnp.float32(N - 1)).astype(jnp.int32)
    targets_f = (ranks + 1).astype(jnp.float32)
    ones16 = jnp.ones((16, 1), dtype=jnp.int32)
    ones_col = jnp.ones((N, 1), dtype=jnp.float32)
    state = []
    for _ in keys:
        state.append(ones16 * LO_KEY)
        state.append(ones16 * HI_KEY)

    def body(_, s):
        out = []
        for j, k in enumerate(keys):
            lo, hi = s[2 * j], s[2 * j + 1]
            mid = lo + ((hi - lo) >> 1)
            mask = (k <= mid).astype(jnp.float32)
            cnt = jnp.dot(mask, ones_col,
                          preferred_element_type=jnp.float32)
            take = cnt >= targets_f
            out.append(jnp.where(take, lo, mid))
            out.append(jnp.where(take, mid, hi))
        return tuple(out)

    final = jax.lax.fori_loop(0, SEARCH_ITERS, body, tuple(state))
    return [jax.lax.bitcast_convert_type(final[2 * j + 1], jnp.float32)
            for j in range(len(keys))]


def _bin_masks(conf, v):
    return (conf > v[0:15, :]) & (conf <= v[1:16, :])


def _ece_kernel(logits_ref, src_ref, lab_ref, out_ref,
                a_ref, lmax_ref, best_ece_ref, best_t_ref):
    pid = pl.program_id(0)

    @pl.when(pid == 0)
    def _source_pass():
        xs = src_ref[:, :]
        xmax = jnp.max(xs, axis=0, keepdims=True)
        e = jnp.exp(xs - xmax)
        z = jnp.sum(e, axis=0, keepdims=True)
        sm = e / z
        conf = jnp.max(sm, axis=0, keepdims=True)
        cls = jax.lax.broadcasted_iota(jnp.int32, (C, N), 0)
        pred = jnp.min(jnp.where(sm == conf, cls, C), axis=0, keepdims=True)
        correct = (pred == lab_ref[:, :]).astype(jnp.float32)
        v = _order_stats(conf)
        mask = _bin_masks(conf, v).astype(jnp.float32)
        cnt = jnp.sum(mask, axis=1, keepdims=True)
        csum = jnp.sum(correct * mask, axis=1, keepdims=True)
        acc = jnp.where(cnt > 0, csum / jnp.maximum(cnt, 1.0), 0.0)
        a_ref[:, :] = jnp.clip(acc, 0.01, 0.99)
        lmax_ref[:, :] = jnp.max(logits_ref[:, :], axis=0, keepdims=True)
        best_ece_ref[:, :] = jnp.full((1, 1), jnp.inf, dtype=jnp.float32)
        best_t_ref[:, :] = jnp.zeros((1, 1), dtype=jnp.float32)
        out_ref[:, :] = jnp.zeros((1, 1), dtype=jnp.float32)

    def _conf_at(t):
        x = logits_ref[:, :] / t
        xmax = lmax_ref[:, :] / t
        z = jnp.sum(jnp.exp(x - xmax), axis=0, keepdims=True)
        conf = 1.0 / z
        return jnp.where(conf == 1.0, jnp.float32(0.999999), conf)

    ece_ones_col = jnp.ones((N, 1), dtype=jnp.float32)

    def _ece_of(conf, v):
        mask = _bin_masks(conf, v).astype(jnp.float32)
        cnt = jnp.dot(mask, ece_ones_col,
                      preferred_element_type=jnp.float32)
        s = jnp.dot(conf * mask, ece_ones_col,
                    preferred_element_type=jnp.float32)
        avgc = s / jnp.maximum(cnt, 1.0)
        term = jnp.where(cnt > 0,
                         jnp.abs(avgc - a_ref[:, :]) * (cnt / jnp.float32(N)),
                         0.0)
        return jnp.sum(term, keepdims=True).reshape(1, 1)

    @pl.when(pid > 0)
    def _temp_pass():
        k = (TEMPS_PER_STEP * (pid - 1)).astype(jnp.float32)
        ts = [jnp.float32(0.5) + jnp.float32(0.01) * (k + j)
              for j in range(TEMPS_PER_STEP)]
        confs = [_conf_at(t) for t in ts]
        vs = _order_stats_n(confs)
        eces = [_ece_of(c, v) for c, v in zip(confs, vs)]
        cur = best_ece_ref[:, :]
        cur_t = best_t_ref[:, :]
        for t, ece in zip(ts, eces):
            b = ece < cur
            cur = jnp.where(b, ece, cur)
            cur_t = jnp.where(b, jnp.full((1, 1), t), cur_t)
        best_ece_ref[:, :] = cur
        best_t_ref[:, :] = cur_t
        out_ref[:, :] = cur_t


@jax.jit
def kernel(logits, source_logits, source_labels):
    logits_t = logits.astype(jnp.float32).T
    src_t = source_logits.astype(jnp.float32).T
    lab = source_labels.astype(jnp.int32).reshape(1, N)
    whole = lambda shape: pl.BlockSpec(shape, lambda i: (0, 0))
    out = pl.pallas_call(
        _ece_kernel,
        grid=(NTEMPS // TEMPS_PER_STEP + 1,),
        in_specs=[whole((C, N)), whole((C, N)), whole((1, N))],
        out_specs=whole((1, 1)),
        out_shape=jax.ShapeDtypeStruct((1, 1), jnp.float32),
        scratch_shapes=[
            pltpu.VMEM((15, 1), jnp.float32),
            pltpu.VMEM((1, N), jnp.float32),
            pltpu.VMEM((1, 1), jnp.float32),
            pltpu.VMEM((1, 1), jnp.float32),
        ],
    )(logits_t, src_t, lab)
    return out.reshape(())


# speedup vs baseline: 3.2289x; 3.2289x over previous
import functools

import jax
import jax.numpy as jnp
from jax.experimental import pallas as pl
from jax.experimental.pallas import tpu as pltpu

N = 10000
C = 100
NTEMPS = 350
RANKS = (0, 666, 1333, 2000, 2666, 3333, 4000, 4666, 5333, 6000,
         6666, 7333, 8000, 8666, 9333, 9999)
LO_KEY = 1008767022
HI_KEY = 1065353216
SEARCH_ITERS = 26
TEMPS_PER_STEP = 10


def _order_stats(conf):
    keys = jax.lax.bitcast_convert_type(conf, jnp.int32)
    idx = jax.lax.broadcasted_iota(jnp.int32, (16, 1), 0).astype(jnp.float32)
    ranks = jnp.minimum(jnp.floor(idx * jnp.float32(10000.0 / 15.0)),
                        jnp.float32(N - 1)).astype(jnp.int32)
    targets = ranks + 1
    ones16 = jnp.ones((16, 1), dtype=jnp.int32)
    lo = ones16 * LO_KEY
    hi = ones16 * HI_KEY

    def body(_, lohi):
        lo, hi = lohi
        mid = lo + ((hi - lo) >> 1)
        cnt = jnp.sum((keys <= mid).astype(jnp.int32), axis=1, keepdims=True)
        take = cnt >= targets
        return jnp.where(take, lo, mid), jnp.where(take, mid, hi)

    _, hi = jax.lax.fori_loop(0, SEARCH_ITERS, body, (lo, hi))
    return jax.lax.bitcast_convert_type(hi, jnp.float32)


def _order_stats_n(confs):
    keys = [jax.lax.bitcast_convert_type(c, jnp.int32) for c in confs]
    idx = jax.lax.broadcasted_iota(jnp.int32, (16, 1), 0).astype(jnp.float32)
    ranks = jnp.minimum(jnp.floor(idx * jnp.float32(10000.0 / 15.0)),
                        jnp.float32(N - 1)).astype(jnp.int32)
    targets = ranks + 1
    ones16 = jnp.ones((16, 1), dtype=jnp.int32)
    state = []
    for _ in keys:
        state.append(ones16 * LO_KEY)
        state.append(ones16 * HI_KEY)

    def body(_, s):
        out = []
        for j, k in enumerate(keys):
            lo, hi = s[2 * j], s[2 * j + 1]
            mid = lo + ((hi - lo) >> 1)
            cnt = jnp.sum((k <= mid).astype(jnp.int32), axis=1, keepdims=True)
            take = cnt >= targets
            out.append(jnp.where(take, lo, mid))
            out.append(jnp.where(take, mid, hi))
        return tuple(out)

    final = jax.lax.fori_loop(0, SEARCH_ITERS, body, tuple(state))
    return [jax.lax.bitcast_convert_type(final[2 * j + 1], jnp.float32)
            for j in range(len(keys))]


def _bin_masks(conf, v):
    return (conf > v[0:15, :]) & (conf <= v[1:16, :])


def _ece_kernel(logits_ref, src_ref, lab_ref, out_ref,
                a_ref, lmax_ref, best_ece_ref, best_t_ref):
    pid = pl.program_id(0)

    @pl.when(pid == 0)
    def _source_pass():
        xs = src_ref[:, :]
        xmax = jnp.max(xs, axis=0, keepdims=True)
        e = jnp.exp(xs - xmax)
        z = jnp.sum(e, axis=0, keepdims=True)
        sm = e / z
        conf = jnp.max(sm, axis=0, keepdims=True)
        cls = jax.lax.broadcasted_iota(jnp.int32, (C, N), 0)
        pred = jnp.min(jnp.where(sm == conf, cls, C), axis=0, keepdims=True)
        correct = (pred == lab_ref[:, :]).astype(jnp.float32)
        v = _order_stats(conf)
        mask = _bin_masks(conf, v).astype(jnp.float32)
        cnt = jnp.sum(mask, axis=1, keepdims=True)
        csum = jnp.sum(correct * mask, axis=1, keepdims=True)
        acc = jnp.where(cnt > 0, csum / jnp.maximum(cnt, 1.0), 0.0)
        a_ref[:, :] = jnp.clip(acc, 0.01, 0.99)
        lmax_ref[:, :] = jnp.max(logits_ref[:, :], axis=0, keepdims=True)
        best_ece_ref[:, :] = jnp.full((1, 1), jnp.inf, dtype=jnp.float32)
        best_t_ref[:, :] = jnp.zeros((1, 1), dtype=jnp.float32)
        out_ref[:, :] = jnp.zeros((1, 1), dtype=jnp.float32)

    def _conf_at(t):
        x = logits_ref[:, :] / t
        xmax = lmax_ref[:, :] / t
        z = jnp.sum(jnp.exp(x - xmax), axis=0, keepdims=True)
        conf = 1.0 / z
        return jnp.where(conf == 1.0, jnp.float32(0.999999), conf)

    def _ece_of(conf, v):
        mask = _bin_masks(conf, v).astype(jnp.float32)
        cnt = jnp.sum(mask, axis=1, keepdims=True)
        s = jnp.sum(conf * mask, axis=1, keepdims=True)
        avgc = s / jnp.maximum(cnt, 1.0)
        term = jnp.where(cnt > 0,
                         jnp.abs(avgc - a_ref[:, :]) * (cnt / jnp.float32(N)),
                         0.0)
        return jnp.sum(term, keepdims=True).reshape(1, 1)

    @pl.when(pid > 0)
    def _temp_pass():
        k = (TEMPS_PER_STEP * (pid - 1)).astype(jnp.float32)
        ts = [jnp.float32(0.5) + jnp.float32(0.01) * (k + j)
              for j in range(TEMPS_PER_STEP)]
        confs = [_conf_at(t) for t in ts]
        vs = _order_stats_n(confs)
        eces = [_ece_of(c, v) for c, v in zip(confs, vs)]
        cur = best_ece_ref[:, :]
        cur_t = best_t_ref[:, :]
        for t, ece in zip(ts, eces):
            b = ece < cur
            cur = jnp.where(b, ece, cur)
            cur_t = jnp.where(b, jnp.full((1, 1), t), cur_t)
        best_ece_ref[:, :] = cur
        best_t_ref[:, :] = cur_t
        out_ref[:, :] = cur_t


@jax.jit
def kernel(logits, source_logits, source_labels):
    logits_t = logits.astype(jnp.float32).T
    src_t = source_logits.astype(jnp.float32).T
    lab = source_labels.astype(jnp.int32).reshape(1, N)
    whole = lambda shape: pl.BlockSpec(shape, lambda i: (0, 0))
    out = pl.pallas_call(
        _ece_kernel,
        grid=(NTEMPS // TEMPS_PER_STEP + 1,),
        in_specs=[whole((C, N)), whole((C, N)), whole((1, N))],
        out_specs=whole((1, 1)),
        out_shape=jax.ShapeDtypeStruct((1, 1), jnp.float32),
        scratch_shapes=[
            pltpu.VMEM((15, 1), jnp.float32),
            pltpu.VMEM((1, N), jnp.float32),
            pltpu.VMEM((1, 1), jnp.float32),
            pltpu.VMEM((1, 1), jnp.float32),
        ],
    )(logits_t, src_t, lab)
    return out.reshape(())


# speedup vs baseline: 3.4570x; 1.0706x over previous
import functools

import jax
import jax.numpy as jnp
from jax.experimental import pallas as pl
from jax.experimental.pallas import tpu as pltpu

N = 10000
C = 100
NTEMPS = 350
RANKS = (0, 666, 1333, 2000, 2666, 3333, 4000, 4666, 5333, 6000,
         6666, 7333, 8000, 8666, 9333, 9999)
LO_KEY = 1008767022
HI_KEY = 1065353216
SEARCH_ITERS = 26
TEMPS_PER_STEP = 10


def _order_stats(conf):
    keys = jax.lax.bitcast_convert_type(conf, jnp.int32)
    idx = jax.lax.broadcasted_iota(jnp.int32, (16, 1), 0).astype(jnp.float32)
    ranks = jnp.minimum(jnp.floor(idx * jnp.float32(10000.0 / 15.0)),
                        jnp.float32(N - 1)).astype(jnp.int32)
    targets = ranks + 1
    ones16 = jnp.ones((16, 1), dtype=jnp.int32)
    lo = ones16 * LO_KEY
    hi = ones16 * HI_KEY

    def body(_, lohi):
        lo, hi = lohi
        mid = lo + ((hi - lo) >> 1)
        cnt = jnp.sum((keys <= mid).astype(jnp.int32), axis=1, keepdims=True)
        take = cnt >= targets
        return jnp.where(take, lo, mid), jnp.where(take, mid, hi)

    _, hi = jax.lax.fori_loop(0, SEARCH_ITERS, body, (lo, hi))
    return jax.lax.bitcast_convert_type(hi, jnp.float32)


def _order_stats_n(confs):
    keys = [jax.lax.bitcast_convert_type(c, jnp.int32) for c in confs]
    idx = jax.lax.broadcasted_iota(jnp.int32, (16, 1), 0).astype(jnp.float32)
    ranks = jnp.minimum(jnp.floor(idx * jnp.float32(10000.0 / 15.0)),
                        jnp.float32(N - 1)).astype(jnp.int32)
    targets = ranks + 1
    ones16 = jnp.ones((16, 1), dtype=jnp.int32)
    state = []
    for _ in keys:
        state.append(ones16 * LO_KEY)
        state.append(ones16 * HI_KEY)

    def body(_, s):
        out = []
        for j, k in enumerate(keys):
            lo, hi = s[2 * j], s[2 * j + 1]
            mid = lo + ((hi - lo) >> 1)
            cnt = jnp.sum((k <= mid).astype(jnp.int32), axis=1, keepdims=True)
            take = cnt >= targets
            out.append(jnp.where(take, lo, mid))
            out.append(jnp.where(take, mid, hi))
        return tuple(out)

    final = jax.lax.fori_loop(0, SEARCH_ITERS, body, tuple(state))
    return [jax.lax.bitcast_convert_type(final[2 * j + 1], jnp.float32)
            for j in range(len(keys))]


def _bin_masks(conf, v):
    return (conf > v[0:15, :]) & (conf <= v[1:16, :])


def _ece_kernel(logits_ref, src_ref, lab_ref, out_ref,
                a_ref, d_ref, best_ece_ref, best_t_ref):
    pid = pl.program_id(0)

    @pl.when(pid == 0)
    def _source_pass():
        xs = src_ref[:, :]
        xmax = jnp.max(xs, axis=0, keepdims=True)
        e = jnp.exp(xs - xmax)
        z = jnp.sum(e, axis=0, keepdims=True)
        sm = e / z
        conf = jnp.max(sm, axis=0, keepdims=True)
        cls = jax.lax.broadcasted_iota(jnp.int32, (C, N), 0)
        pred = jnp.min(jnp.where(sm == conf, cls, C), axis=0, keepdims=True)
        correct = (pred == lab_ref[:, :]).astype(jnp.float32)
        v = _order_stats(conf)
        mask = _bin_masks(conf, v).astype(jnp.float32)
        cnt = jnp.sum(mask, axis=1, keepdims=True)
        csum = jnp.sum(correct * mask, axis=1, keepdims=True)
        acc = jnp.where(cnt > 0, csum / jnp.maximum(cnt, 1.0), 0.0)
        a_ref[:, :] = jnp.clip(acc, 0.01, 0.99)
        lg = logits_ref[:, :]
        d_ref[:, :] = lg - jnp.max(lg, axis=0, keepdims=True)
        best_ece_ref[:, :] = jnp.full((1, 1), jnp.inf, dtype=jnp.float32)
        best_t_ref[:, :] = jnp.zeros((1, 1), dtype=jnp.float32)
        out_ref[:, :] = jnp.zeros((1, 1), dtype=jnp.float32)

    def _conf_at(t):
        rinv = 1.0 / t
        z = jnp.sum(jnp.exp(d_ref[:, :] * rinv), axis=0, keepdims=True)
        conf = 1.0 / z
        return jnp.where(conf == 1.0, jnp.float32(0.999999), conf)

    def _ece_of(conf, v):
        mask = _bin_masks(conf, v).astype(jnp.float32)
        cnt = jnp.sum(mask, axis=1, keepdims=True)
        s = jnp.sum(conf * mask, axis=1, keepdims=True)
        avgc = s / jnp.maximum(cnt, 1.0)
        term = jnp.where(cnt > 0,
                         jnp.abs(avgc - a_ref[:, :]) * (cnt / jnp.float32(N)),
                         0.0)
        return jnp.sum(term, keepdims=True).reshape(1, 1)

    @pl.when(pid > 0)
    def _temp_pass():
        k = (TEMPS_PER_STEP * (pid - 1)).astype(jnp.float32)
        ts = [jnp.float32(0.5) + jnp.float32(0.01) * (k + j)
              for j in range(TEMPS_PER_STEP)]
        confs = [_conf_at(t) for t in ts]
        vs = _order_stats_n(confs)
        eces = [_ece_of(c, v) for c, v in zip(confs, vs)]
        cur = best_ece_ref[:, :]
        cur_t = best_t_ref[:, :]
        for t, ece in zip(ts, eces):
            b = ece < cur
            cur = jnp.where(b, ece, cur)
            cur_t = jnp.where(b, jnp.full((1, 1), t), cur_t)
        best_ece_ref[:, :] = cur
        best_t_ref[:, :] = cur_t
        out_ref[:, :] = cur_t


@jax.jit
def kernel(logits, source_logits, source_labels):
    logits_t = logits.astype(jnp.float32).T
    src_t = source_logits.astype(jnp.float32).T
    lab = source_labels.astype(jnp.int32).reshape(1, N)
    whole = lambda shape: pl.BlockSpec(shape, lambda i: (0, 0))
    out = pl.pallas_call(
        _ece_kernel,
        grid=(NTEMPS // TEMPS_PER_STEP + 1,),
        in_specs=[whole((C, N)), whole((C, N)), whole((1, N))],
        out_specs=whole((1, 1)),
        out_shape=jax.ShapeDtypeStruct((1, 1), jnp.float32),
        scratch_shapes=[
            pltpu.VMEM((15, 1), jnp.float32),
            pltpu.VMEM((C, N), jnp.float32),
            pltpu.VMEM((1, 1), jnp.float32),
            pltpu.VMEM((1, 1), jnp.float32),
        ],
    )(logits_t, src_t, lab)
    return out.reshape(())


# speedup vs baseline: 3.5873x; 1.0377x over previous
import functools

import jax
import jax.numpy as jnp
from jax.experimental import pallas as pl
from jax.experimental.pallas import tpu as pltpu

N = 10000
C = 100
NTEMPS = 350
RANKS = (0, 666, 1333, 2000, 2666, 3333, 4000, 4666, 5333, 6000,
         6666, 7333, 8000, 8666, 9333, 9999)
LO_KEY = 1008767022
HI_KEY = 1065353216
SEARCH_ITERS = 26
TEMPS_PER_STEP = 14


def _order_stats(conf):
    keys = jax.lax.bitcast_convert_type(conf, jnp.int32)
    idx = jax.lax.broadcasted_iota(jnp.int32, (16, 1), 0).astype(jnp.float32)
    ranks = jnp.minimum(jnp.floor(idx * jnp.float32(10000.0 / 15.0)),
                        jnp.float32(N - 1)).astype(jnp.int32)
    targets = ranks + 1
    ones16 = jnp.ones((16, 1), dtype=jnp.int32)
    lo = ones16 * LO_KEY
    hi = ones16 * HI_KEY

    def body(_, lohi):
        lo, hi = lohi
        mid = lo + ((hi - lo) >> 1)
        cnt = jnp.sum((keys <= mid).astype(jnp.int32), axis=1, keepdims=True)
        take = cnt >= targets
        return jnp.where(take, lo, mid), jnp.where(take, mid, hi)

    _, hi = jax.lax.fori_loop(0, SEARCH_ITERS, body, (lo, hi))
    return jax.lax.bitcast_convert_type(hi, jnp.float32)


def _order_stats_n(confs):
    keys = [jax.lax.bitcast_convert_type(c, jnp.int32) for c in confs]
    idx = jax.lax.broadcasted_iota(jnp.int32, (16, 1), 0).astype(jnp.float32)
    ranks = jnp.minimum(jnp.floor(idx * jnp.float32(10000.0 / 15.0)),
                        jnp.float32(N - 1)).astype(jnp.int32)
    targets = ranks + 1
    ones16 = jnp.ones((16, 1), dtype=jnp.int32)
    state = []
    for _ in keys:
        state.append(ones16 * LO_KEY)
        state.append(ones16 * HI_KEY)

    def body(_, s):
        out = []
        for j, k in enumerate(keys):
            lo, hi = s[2 * j], s[2 * j + 1]
            mid = lo + ((hi - lo) >> 1)
            cnt = jnp.sum((k <= mid).astype(jnp.int32), axis=1, keepdims=True)
            take = cnt >= targets
            out.append(jnp.where(take, lo, mid))
            out.append(jnp.where(take, mid, hi))
        return tuple(out)

    final = jax.lax.fori_loop(0, SEARCH_ITERS, body, tuple(state))
    return [jax.lax.bitcast_convert_type(final[2 * j + 1], jnp.float32)
            for j in range(len(keys))]


def _bin_masks(conf, v):
    return (conf > v[0:15, :]) & (conf <= v[1:16, :])


def _ece_kernel(logits_ref, src_ref, lab_ref, out_ref,
                a_ref, d_ref, best_ece_ref, best_t_ref):
    pid = pl.program_id(0)

    @pl.when(pid == 0)
    def _source_pass():
        xs = src_ref[:, :]
        xmax = jnp.max(xs, axis=0, keepdims=True)
        e = jnp.exp(xs - xmax)
        z = jnp.sum(e, axis=0, keepdims=True)
        sm = e / z
        conf = jnp.max(sm, axis=0, keepdims=True)
        cls = jax.lax.broadcasted_iota(jnp.int32, (C, N), 0)
        pred = jnp.min(jnp.where(sm == conf, cls, C), axis=0, keepdims=True)
        correct = (pred == lab_ref[:, :]).astype(jnp.float32)
        v = _order_stats(conf)
        mask = _bin_masks(conf, v).astype(jnp.float32)
        cnt = jnp.sum(mask, axis=1, keepdims=True)
        csum = jnp.sum(correct * mask, axis=1, keepdims=True)
        acc = jnp.where(cnt > 0, csum / jnp.maximum(cnt, 1.0), 0.0)
        a_ref[:, :] = jnp.clip(acc, 0.01, 0.99)
        lg = logits_ref[:, :]
        d_ref[:, :] = lg - jnp.max(lg, axis=0, keepdims=True)
        best_ece_ref[:, :] = jnp.full((1, 1), jnp.inf, dtype=jnp.float32)
        best_t_ref[:, :] = jnp.zeros((1, 1), dtype=jnp.float32)
        out_ref[:, :] = jnp.zeros((1, 1), dtype=jnp.float32)

    def _conf_at(t):
        rinv = 1.0 / t
        z = jnp.sum(jnp.exp(d_ref[:, :] * rinv), axis=0, keepdims=True)
        conf = 1.0 / z
        return jnp.where(conf == 1.0, jnp.float32(0.999999), conf)

    def _ece_of(conf, v):
        mask = _bin_masks(conf, v).astype(jnp.float32)
        cnt = jnp.sum(mask, axis=1, keepdims=True)
        s = jnp.sum(conf * mask, axis=1, keepdims=True)
        avgc = s / jnp.maximum(cnt, 1.0)
        term = jnp.where(cnt > 0,
                         jnp.abs(avgc - a_ref[:, :]) * (cnt / jnp.float32(N)),
                         0.0)
        return jnp.sum(term, keepdims=True).reshape(1, 1)

    @pl.when(pid > 0)
    def _temp_pass():
        k = (TEMPS_PER_STEP * (pid - 1)).astype(jnp.float32)
        ts = [jnp.float32(0.5) + jnp.float32(0.01) * (k + j)
              for j in range(TEMPS_PER_STEP)]
        confs = [_conf_at(t) for t in ts]
        vs = _order_stats_n(confs)
        eces = [_ece_of(c, v) for c, v in zip(confs, vs)]
        cur = best_ece_ref[:, :]
        cur_t = best_t_ref[:, :]
        for t, ece in zip(ts, eces):
            b = ece < cur
            cur = jnp.where(b, ece, cur)
            cur_t = jnp.where(b, jnp.full((1, 1), t), cur_t)
        best_ece_ref[:, :] = cur
        best_t_ref[:, :] = cur_t
        out_ref[:, :] = cur_t


@jax.jit
def kernel(logits, source_logits, source_labels):
    logits_t = logits.astype(jnp.float32).T
    src_t = source_logits.astype(jnp.float32).T
    lab = source_labels.astype(jnp.int32).reshape(1, N)
    whole = lambda shape: pl.BlockSpec(shape, lambda i: (0, 0))
    out = pl.pallas_call(
        _ece_kernel,
        grid=(NTEMPS // TEMPS_PER_STEP + 1,),
        in_specs=[whole((C, N)), whole((C, N)), whole((1, N))],
        out_specs=whole((1, 1)),
        out_shape=jax.ShapeDtypeStruct((1, 1), jnp.float32),
        scratch_shapes=[
            pltpu.VMEM((15, 1), jnp.float32),
            pltpu.VMEM((C, N), jnp.float32),
            pltpu.VMEM((1, 1), jnp.float32),
            pltpu.VMEM((1, 1), jnp.float32),
        ],
    )(logits_t, src_t, lab)
    return out.reshape(())
